# baseline (speedup 1.0000x reference)
import math

import jax
import jax.numpy as jnp
from jax.experimental import pallas as pl
from jax.experimental.pallas import tpu as pltpu

_B, _T, _M, _DIN = 50, 500, 4, 128
_D, _H = 128, 4
_C = _D // _H
_BT = _B * _T
_FB = 200


def _lane(x, m):
    return x[:, m * _D:(m + 1) * _D]


def _cat(a, b):
    return jnp.concatenate([a, b], axis=1)


_PAIRS = [(0, 1), (0, 2), (0, 3), (1, 0), (1, 2), (1, 3),
          (2, 0), (2, 1), (2, 3), (3, 0), (3, 1), (3, 2)]


def _attn_layer(hs, Wc, bc, u, w, gamma, beta, S2, J2):
    qkvr = [jnp.dot(h, Wc, preferred_element_type=jnp.float32) + bc
            for h in hs]
    q = [_lane(t, 0) for t in qkvr]
    k = [_lane(t, 1) for t in qkvr]
    v = [_lane(t, 2) for t in qkvr]
    r = [_lane(t, 3) for t in qkvr]

    e = {}
    for t in range(6):
        (i0, j0), (i1, j1) = _PAIRS[2 * t], _PAIRS[2 * t + 1]
        prod = _cat(q[i0] * k[j0], q[i1] * k[j1])
        a2 = jnp.dot(prod, S2, preferred_element_type=jnp.float32)
        ee = jnp.exp(a2)
        e[(i0, j0)] = ee[:, :_D]
        e[(i1, j1)] = ee[:, _D:]

    out = []
    for i in range(_M):
        js = [j for j in range(_M) if j != i]
        rden = 1.0 / (e[(i, js[0])] + e[(i, js[1])] + e[(i, js[2])] + 1e-16)
        out.append((e[(i, js[0])] * v[js[0]] + e[(i, js[1])] * v[js[1]]
                    + e[(i, js[2])] * v[js[2]]) * rden)

    gin = [out[i] * u + r[i] * w for i in range(_M)]
    gp01 = jnp.dot(_cat(gin[0], gin[1]), J2, preferred_element_type=jnp.float32)
    gp23 = jnp.dot(_cat(gin[2], gin[3]), J2, preferred_element_type=jnp.float32)
    gp = [gp01[:, :_D], gp01[:, _D:], gp23[:, :_D], gp23[:, _D:]]

    hh = [None] * _M
    for i in range(_M):
        gt = jax.nn.sigmoid(gp[i])
        hh[i] = gt * r[i] + (1.0 - gt) * out[i]

    mu01 = jnp.dot(_cat(hh[0], hh[1]), J2, preferred_element_type=jnp.float32)
    mu23 = jnp.dot(_cat(hh[2], hh[3]), J2, preferred_element_type=jnp.float32)
    s201 = jnp.dot(_cat(hh[0] * hh[0], hh[1] * hh[1]), J2,
                   preferred_element_type=jnp.float32)
    s223 = jnp.dot(_cat(hh[2] * hh[2], hh[3] * hh[3]), J2,
                   preferred_element_type=jnp.float32)
    mu = [mu01[:, :_D], mu01[:, _D:], mu23[:, :_D], mu23[:, _D:]]
    s2 = [s201[:, :_D], s201[:, _D:], s223[:, :_D], s223[:, _D:]]

    hn = []
    for i in range(_M):
        m_i = mu[i] * (1.0 / _D)
        var = s2[i] * (1.0 / _D) - m_i * m_i
        hn.append(jnp.maximum(
            (hh[i] - m_i) * jax.lax.rsqrt(var + 1e-5) * gamma + beta, 0.0))
    return hn


def _gnn_kernel(x_ref, E2_ref, bemb2_ref,
                W1_ref, b1_ref, u1_ref, w1_ref, g1_ref, be1_ref,
                W2_ref, b2_ref, u2_ref, w2_ref, g2_ref, be2_ref,
                S2_ref, J2_ref, out_ref):
    S2 = S2_ref[...]
    J2 = J2_ref[...]
    E2 = E2_ref[...]
    bemb2 = bemb2_ref[...]
    x = x_ref[...]
    h01 = jnp.maximum(jnp.dot(x[:, :2 * _D], E2,
                              preferred_element_type=jnp.float32) + bemb2, 0.0)
    h23 = jnp.maximum(jnp.dot(x[:, 2 * _D:], E2,
                              preferred_element_type=jnp.float32) + bemb2, 0.0)
    hs = [h01[:, :_D], h01[:, _D:], h23[:, :_D], h23[:, _D:]]
    hs = _attn_layer(hs, W1_ref[...], b1_ref[...], u1_ref[...], w1_ref[...],
                     g1_ref[...], be1_ref[...], S2, J2)
    hs = _attn_layer(hs, W2_ref[...], b2_ref[...], u2_ref[...], w2_ref[...],
                     g2_ref[...], be2_ref[...], S2, J2)
    for m in range(_M):
        out_ref[:, m * _D:(m + 1) * _D] = hs[m]


def kernel(x, W_emb, b_emb,
           Wq1, bq1, Wk1, bk1, Wv1, bv1, Ws1, bs1, Wb1, g1, be1,
           Wq2, bq2, Wk2, bk2, Wv2, bv2, Ws2, bs2, Wb2, g2, be2):
    Bx, Tx, Mx, Dx = x.shape
    BT = Bx * Tx
    xf = x.reshape(BT, Mx * Dx)

    W1 = jnp.concatenate([Wq1, Wk1, Wv1, Ws1], axis=1)
    b1 = jnp.concatenate([bq1, bk1, bv1, bs1]).reshape(1, 4 * _D)
    W2 = jnp.concatenate([Wq2, Wk2, Wv2, Ws2], axis=1)
    b2 = jnp.concatenate([bq2, bk2, bv2, bs2]).reshape(1, 4 * _D)
    u1 = (Wb1[0:_D, 0] + Wb1[2 * _D:3 * _D, 0]).reshape(1, _D)
    w1 = (Wb1[_D:2 * _D, 0] - Wb1[2 * _D:3 * _D, 0]).reshape(1, _D)
    u2 = (Wb2[0:_D, 0] + Wb2[2 * _D:3 * _D, 0]).reshape(1, _D)
    w2 = (Wb2[_D:2 * _D, 0] - Wb2[2 * _D:3 * _D, 0]).reshape(1, _D)

    S = jnp.kron(jnp.eye(_H, dtype=jnp.float32),
                 jnp.full((_C, _C), 1.0 / math.sqrt(_C), dtype=jnp.float32))
    eye2 = jnp.eye(2, dtype=jnp.float32)
    S2 = jnp.kron(eye2, S)
    J2 = jnp.kron(eye2, jnp.ones((_D, _D), dtype=jnp.float32))
    E2 = jnp.kron(eye2, W_emb)
    bemb2 = jnp.concatenate([b_emb, b_emb]).reshape(1, 2 * _D)

    wspec = lambda shape: pl.BlockSpec(shape, lambda i: (0, 0))
    grid = BT // _FB

    out = pl.pallas_call(
        _gnn_kernel,
        grid=(grid,),
        in_specs=[
            pl.BlockSpec((_FB, Mx * Dx), lambda i: (i, 0)),
            wspec((2 * Dx, 2 * _D)), wspec((1, 2 * _D)),
            wspec((_D, 4 * _D)), wspec((1, 4 * _D)), wspec((1, _D)),
            wspec((1, _D)), wspec((1, _D)), wspec((1, _D)),
            wspec((_D, 4 * _D)), wspec((1, 4 * _D)), wspec((1, _D)),
            wspec((1, _D)), wspec((1, _D)), wspec((1, _D)),
            wspec((2 * _D, 2 * _D)), wspec((2 * _D, 2 * _D)),
        ],
        out_specs=pl.BlockSpec((_FB, Mx * _D), lambda i: (i, 0)),
        out_shape=jax.ShapeDtypeStruct((BT, Mx * _D), jnp.float32),
        compiler_params=pltpu.CompilerParams(
            dimension_semantics=("parallel",)),
    )(xf, E2, bemb2,
      W1, b1, u1, w1, g1.reshape(1, _D), be1.reshape(1, _D),
      W2, b2, u2, w2, g2.reshape(1, _D), be2.reshape(1, _D),
      S2, J2)

    return out.reshape(Bx, Tx, Mx, _D)


# speedup vs baseline: 1.0717x; 1.0717x over previous
import math

import jax
import jax.numpy as jnp
from jax.experimental import pallas as pl
from jax.experimental.pallas import tpu as pltpu

_B, _T, _M, _DIN = 50, 500, 4, 128
_D, _H = 128, 4
_C = _D // _H
_BT = _B * _T
_FB = 1000


def _lane(x, m):
    return x[:, m * _D:(m + 1) * _D]


def _cat(a, b):
    return jnp.concatenate([a, b], axis=1)


_PAIRS = [(0, 1), (0, 2), (0, 3), (1, 0), (1, 2), (1, 3),
          (2, 0), (2, 1), (2, 3), (3, 0), (3, 1), (3, 2)]


def _attn_layer(hs, Wc, bc, u, w, gamma, beta, S2, J2):
    qkvr = [jnp.dot(h, Wc, preferred_element_type=jnp.float32) + bc
            for h in hs]
    q = [_lane(t, 0) for t in qkvr]
    k = [_lane(t, 1) for t in qkvr]
    v = [_lane(t, 2) for t in qkvr]
    r = [_lane(t, 3) for t in qkvr]

    e = {}
    for t in range(6):
        (i0, j0), (i1, j1) = _PAIRS[2 * t], _PAIRS[2 * t + 1]
        prod = _cat(q[i0] * k[j0], q[i1] * k[j1])
        a2 = jnp.dot(prod, S2, preferred_element_type=jnp.float32)
        ee = jnp.exp(a2)
        e[(i0, j0)] = ee[:, :_D]
        e[(i1, j1)] = ee[:, _D:]

    out = []
    for i in range(_M):
        js = [j for j in range(_M) if j != i]
        rden = 1.0 / (e[(i, js[0])] + e[(i, js[1])] + e[(i, js[2])] + 1e-16)
        out.append((e[(i, js[0])] * v[js[0]] + e[(i, js[1])] * v[js[1]]
                    + e[(i, js[2])] * v[js[2]]) * rden)

    gin = [out[i] * u + r[i] * w for i in range(_M)]
    gp01 = jnp.dot(_cat(gin[0], gin[1]), J2, preferred_element_type=jnp.float32)
    gp23 = jnp.dot(_cat(gin[2], gin[3]), J2, preferred_element_type=jnp.float32)
    gp = [gp01[:, :_D], gp01[:, _D:], gp23[:, :_D], gp23[:, _D:]]

    hh = [None] * _M
    for i in range(_M):
        gt = jax.nn.sigmoid(gp[i])
        hh[i] = gt * r[i] + (1.0 - gt) * out[i]

    mu01 = jnp.dot(_cat(hh[0], hh[1]), J2, preferred_element_type=jnp.float32)
    mu23 = jnp.dot(_cat(hh[2], hh[3]), J2, preferred_element_type=jnp.float32)
    s201 = jnp.dot(_cat(hh[0] * hh[0], hh[1] * hh[1]), J2,
                   preferred_element_type=jnp.float32)
    s223 = jnp.dot(_cat(hh[2] * hh[2], hh[3] * hh[3]), J2,
                   preferred_element_type=jnp.float32)
    mu = [mu01[:, :_D], mu01[:, _D:], mu23[:, :_D], mu23[:, _D:]]
    s2 = [s201[:, :_D], s201[:, _D:], s223[:, :_D], s223[:, _D:]]

    hn = []
    for i in range(_M):
        m_i = mu[i] * (1.0 / _D)
        var = s2[i] * (1.0 / _D) - m_i * m_i
        hn.append(jnp.maximum(
            (hh[i] - m_i) * jax.lax.rsqrt(var + 1e-5) * gamma + beta, 0.0))
    return hn


def _gnn_kernel(x_ref, E2_ref, bemb2_ref,
                W1_ref, b1_ref, u1_ref, w1_ref, g1_ref, be1_ref,
                W2_ref, b2_ref, u2_ref, w2_ref, g2_ref, be2_ref,
                S2_ref, J2_ref, out_ref):
    S2 = S2_ref[...]
    J2 = J2_ref[...]
    E2 = E2_ref[...]
    bemb2 = bemb2_ref[...]
    x = x_ref[...]
    h01 = jnp.maximum(jnp.dot(x[:, :2 * _D], E2,
                              preferred_element_type=jnp.float32) + bemb2, 0.0)
    h23 = jnp.maximum(jnp.dot(x[:, 2 * _D:], E2,
                              preferred_element_type=jnp.float32) + bemb2, 0.0)
    hs = [h01[:, :_D], h01[:, _D:], h23[:, :_D], h23[:, _D:]]
    hs = _attn_layer(hs, W1_ref[...], b1_ref[...], u1_ref[...], w1_ref[...],
                     g1_ref[...], be1_ref[...], S2, J2)
    hs = _attn_layer(hs, W2_ref[...], b2_ref[...], u2_ref[...], w2_ref[...],
                     g2_ref[...], be2_ref[...], S2, J2)
    for m in range(_M):
        out_ref[:, m * _D:(m + 1) * _D] = hs[m]


def kernel(x, W_emb, b_emb,
           Wq1, bq1, Wk1, bk1, Wv1, bv1, Ws1, bs1, Wb1, g1, be1,
           Wq2, bq2, Wk2, bk2, Wv2, bv2, Ws2, bs2, Wb2, g2, be2):
    Bx, Tx, Mx, Dx = x.shape
    BT = Bx * Tx
    xf = x.reshape(BT, Mx * Dx)

    W1 = jnp.concatenate([Wq1, Wk1, Wv1, Ws1], axis=1)
    b1 = jnp.concatenate([bq1, bk1, bv1, bs1]).reshape(1, 4 * _D)
    W2 = jnp.concatenate([Wq2, Wk2, Wv2, Ws2], axis=1)
    b2 = jnp.concatenate([bq2, bk2, bv2, bs2]).reshape(1, 4 * _D)
    u1 = (Wb1[0:_D, 0] + Wb1[2 * _D:3 * _D, 0]).reshape(1, _D)
    w1 = (Wb1[_D:2 * _D, 0] - Wb1[2 * _D:3 * _D, 0]).reshape(1, _D)
    u2 = (Wb2[0:_D, 0] + Wb2[2 * _D:3 * _D, 0]).reshape(1, _D)
    w2 = (Wb2[_D:2 * _D, 0] - Wb2[2 * _D:3 * _D, 0]).reshape(1, _D)

    S = jnp.kron(jnp.eye(_H, dtype=jnp.float32),
                 jnp.full((_C, _C), 1.0 / math.sqrt(_C), dtype=jnp.float32))
    eye2 = jnp.eye(2, dtype=jnp.float32)
    S2 = jnp.kron(eye2, S)
    J2 = jnp.kron(eye2, jnp.ones((_D, _D), dtype=jnp.float32))
    E2 = jnp.kron(eye2, W_emb)
    bemb2 = jnp.concatenate([b_emb, b_emb]).reshape(1, 2 * _D)

    wspec = lambda shape: pl.BlockSpec(shape, lambda i: (0, 0))
    grid = BT // _FB

    out = pl.pallas_call(
        _gnn_kernel,
        grid=(grid,),
        in_specs=[
            pl.BlockSpec((_FB, Mx * Dx), lambda i: (i, 0)),
            wspec((2 * Dx, 2 * _D)), wspec((1, 2 * _D)),
            wspec((_D, 4 * _D)), wspec((1, 4 * _D)), wspec((1, _D)),
            wspec((1, _D)), wspec((1, _D)), wspec((1, _D)),
            wspec((_D, 4 * _D)), wspec((1, 4 * _D)), wspec((1, _D)),
            wspec((1, _D)), wspec((1, _D)), wspec((1, _D)),
            wspec((2 * _D, 2 * _D)), wspec((2 * _D, 2 * _D)),
        ],
        out_specs=pl.BlockSpec((_FB, Mx * _D), lambda i: (i, 0)),
        out_shape=jax.ShapeDtypeStruct((BT, Mx * _D), jnp.float32),
        compiler_params=pltpu.CompilerParams(
            dimension_semantics=("parallel",)),
    )(xf, E2, bemb2,
      W1, b1, u1, w1, g1.reshape(1, _D), be1.reshape(1, _D),
      W2, b2, u2, w2, g2.reshape(1, _D), be2.reshape(1, _D),
      S2, J2)

    return out.reshape(Bx, Tx, Mx, _D)


# speedup vs baseline: 1.3017x; 1.2146x over previous
import math

import jax
import jax.numpy as jnp
from jax.experimental import pallas as pl
from jax.experimental.pallas import tpu as pltpu

_B, _T, _M, _DIN = 50, 500, 4, 128
_D, _H = 128, 4
_C = _D // _H
_BT = _B * _T
_FB = 1000


def _lane(x, m):
    return x[:, m * _D:(m + 1) * _D]


def _cat(a, b):
    return jnp.concatenate([a, b], axis=1)


_PAIRS = [(0, 1), (0, 2), (0, 3), (1, 0), (1, 2), (1, 3),
          (2, 0), (2, 1), (2, 3), (3, 0), (3, 1), (3, 2)]


def _attn_layer(hs, Wc, bc, u, w, gamma, beta, S2, J2):
    qkvr = [jnp.dot(h, Wc, preferred_element_type=jnp.float32) + bc
            for h in hs]
    q = [_lane(t, 0) for t in qkvr]
    k = [_lane(t, 1) for t in qkvr]
    v = [_lane(t, 2) for t in qkvr]
    r = [_lane(t, 3) for t in qkvr]

    e = {}
    for t in range(6):
        (i0, j0), (i1, j1) = _PAIRS[2 * t], _PAIRS[2 * t + 1]
        prod = _cat(q[i0] * k[j0], q[i1] * k[j1])
        a2 = jnp.dot(prod, S2, preferred_element_type=jnp.float32)
        ee = jnp.exp(a2)
        e[(i0, j0)] = ee[:, :_D]
        e[(i1, j1)] = ee[:, _D:]

    out = []
    for i in range(_M):
        js = [j for j in range(_M) if j != i]
        rden = 1.0 / (e[(i, js[0])] + e[(i, js[1])] + e[(i, js[2])] + 1e-16)
        out.append((e[(i, js[0])] * v[js[0]] + e[(i, js[1])] * v[js[1]]
                    + e[(i, js[2])] * v[js[2]]) * rden)

    gin = [out[i] * u + r[i] * w for i in range(_M)]
    gp01 = jnp.dot(_cat(gin[0], gin[1]), J2, preferred_element_type=jnp.float32)
    gp23 = jnp.dot(_cat(gin[2], gin[3]), J2, preferred_element_type=jnp.float32)
    gp = [gp01[:, :_D], gp01[:, _D:], gp23[:, :_D], gp23[:, _D:]]

    hh = [None] * _M
    for i in range(_M):
        gt = jax.nn.sigmoid(gp[i])
        hh[i] = gt * r[i] + (1.0 - gt) * out[i]

    mu01 = jnp.dot(_cat(hh[0], hh[1]), J2, preferred_element_type=jnp.float32)
    mu23 = jnp.dot(_cat(hh[2], hh[3]), J2, preferred_element_type=jnp.float32)
    s201 = jnp.dot(_cat(hh[0] * hh[0], hh[1] * hh[1]), J2,
                   preferred_element_type=jnp.float32)
    s223 = jnp.dot(_cat(hh[2] * hh[2], hh[3] * hh[3]), J2,
                   preferred_element_type=jnp.float32)
    mu = [mu01[:, :_D], mu01[:, _D:], mu23[:, :_D], mu23[:, _D:]]
    s2 = [s201[:, :_D], s201[:, _D:], s223[:, :_D], s223[:, _D:]]

    hn = []
    for i in range(_M):
        m_i = mu[i] * (1.0 / _D)
        var = s2[i] * (1.0 / _D) - m_i * m_i
        hn.append(jnp.maximum(
            (hh[i] - m_i) * jax.lax.rsqrt(var + 1e-5) * gamma + beta, 0.0))
    return hn


def _gnn_kernel(x_ref, E2_ref, bemb2_ref,
                W1_ref, b1_ref, u1_ref, w1_ref, g1_ref, be1_ref,
                W2_ref, b2_ref, u2_ref, w2_ref, g2_ref, be2_ref,
                S2_ref, J2_ref, out_ref):
    S2 = S2_ref[...]
    J2 = J2_ref[...]
    E2 = E2_ref[...]
    bemb2 = bemb2_ref[...]
    x4 = x_ref[...]
    xm = [x4[:, m, :] for m in range(_M)]
    h01 = jnp.maximum(jnp.dot(_cat(xm[0], xm[1]), E2,
                              preferred_element_type=jnp.float32) + bemb2, 0.0)
    h23 = jnp.maximum(jnp.dot(_cat(xm[2], xm[3]), E2,
                              preferred_element_type=jnp.float32) + bemb2, 0.0)
    hs = [h01[:, :_D], h01[:, _D:], h23[:, :_D], h23[:, _D:]]
    hs = _attn_layer(hs, W1_ref[...], b1_ref[...], u1_ref[...], w1_ref[...],
                     g1_ref[...], be1_ref[...], S2, J2)
    hs = _attn_layer(hs, W2_ref[...], b2_ref[...], u2_ref[...], w2_ref[...],
                     g2_ref[...], be2_ref[...], S2, J2)
    for m in range(_M):
        out_ref[:, m, :] = hs[m]


def kernel(x, W_emb, b_emb,
           Wq1, bq1, Wk1, bk1, Wv1, bv1, Ws1, bs1, Wb1, g1, be1,
           Wq2, bq2, Wk2, bk2, Wv2, bv2, Ws2, bs2, Wb2, g2, be2):
    Bx, Tx, Mx, Dx = x.shape
    BT = Bx * Tx
    xf = x.reshape(BT, Mx, Dx)

    W1 = jnp.concatenate([Wq1, Wk1, Wv1, Ws1], axis=1)
    b1 = jnp.concatenate([bq1, bk1, bv1, bs1]).reshape(1, 4 * _D)
    W2 = jnp.concatenate([Wq2, Wk2, Wv2, Ws2], axis=1)
    b2 = jnp.concatenate([bq2, bk2, bv2, bs2]).reshape(1, 4 * _D)
    u1 = (Wb1[0:_D, 0] + Wb1[2 * _D:3 * _D, 0]).reshape(1, _D)
    w1 = (Wb1[_D:2 * _D, 0] - Wb1[2 * _D:3 * _D, 0]).reshape(1, _D)
    u2 = (Wb2[0:_D, 0] + Wb2[2 * _D:3 * _D, 0]).reshape(1, _D)
    w2 = (Wb2[_D:2 * _D, 0] - Wb2[2 * _D:3 * _D, 0]).reshape(1, _D)

    S = jnp.kron(jnp.eye(_H, dtype=jnp.float32),
                 jnp.full((_C, _C), 1.0 / math.sqrt(_C), dtype=jnp.float32))
    eye2 = jnp.eye(2, dtype=jnp.float32)
    S2 = jnp.kron(eye2, S)
    J2 = jnp.kron(eye2, jnp.ones((_D, _D), dtype=jnp.float32))
    E2 = jnp.kron(eye2, W_emb)
    bemb2 = jnp.concatenate([b_emb, b_emb]).reshape(1, 2 * _D)

    wspec = lambda shape: pl.BlockSpec(shape, lambda i: (0, 0))
    grid = BT // _FB

    out = pl.pallas_call(
        _gnn_kernel,
        grid=(grid,),
        in_specs=[
            pl.BlockSpec((_FB, Mx, Dx), lambda i: (i, 0, 0)),
            wspec((2 * Dx, 2 * _D)), wspec((1, 2 * _D)),
            wspec((_D, 4 * _D)), wspec((1, 4 * _D)), wspec((1, _D)),
            wspec((1, _D)), wspec((1, _D)), wspec((1, _D)),
            wspec((_D, 4 * _D)), wspec((1, 4 * _D)), wspec((1, _D)),
            wspec((1, _D)), wspec((1, _D)), wspec((1, _D)),
            wspec((2 * _D, 2 * _D)), wspec((2 * _D, 2 * _D)),
        ],
        out_specs=pl.BlockSpec((_FB, Mx, _D), lambda i: (i, 0, 0)),
        out_shape=jax.ShapeDtypeStruct((BT, Mx, _D), jnp.float32),
        compiler_params=pltpu.CompilerParams(
            dimension_semantics=("parallel",)),
    )(xf, E2, bemb2,
      W1, b1, u1, w1, g1.reshape(1, _D), be1.reshape(1, _D),
      W2, b2, u2, w2, g2.reshape(1, _D), be2.reshape(1, _D),
      S2, J2)

    return out.reshape(Bx, Tx, Mx, _D)
